# baseline (speedup 1.0000x reference)
import functools

import jax
import jax.numpy as jnp
from jax import lax
from jax.experimental import pallas as pl
from jax.experimental.pallas import tpu as pltpu
from jax.experimental.pallas import tpu_sc as plsc

N = 10000
E = 320000
D = 128
G = 256
L = 5

NC = 2
NS = 16
CHUNK = 128
NW = NC * NS
CPW = 80
HCPW = CPW // 2
NPH = 2
E2 = NW * CPW * CHUNK
NCH = E2 // CHUNK
NPAD = N + 16
RPS = 624
TAIL = N - NS * RPS
ZTAIL = NPAD - NS * RPS
TAIL_OFF = NS * RPS
ZR = 208

BLK = 2000
_PREC = lax.Precision.HIGHEST


def _dotT(a, w):
    return lax.dot_general(a, w, (((1,), (1,)), ((), ())), precision=_PREC)


def _extract_row(blk, j, out1d):
    for t in range(CHUNK // 16):
        out1d[pl.ds(16 * t, 16)] = blk[j, pl.ds(16 * t, 16)]


def _sc_mesh():
    return plsc.VectorSubcoreMesh(core_axis_name="c", subcore_axis_name="s")


def _deg_sc(dst2d):
    @functools.partial(
        pl.kernel,
        out_type=jax.ShapeDtypeStruct((NC, N, 16), jnp.float32),
        mesh=_sc_mesh(),
        scratch_types=[
            pltpu.VMEM((CHUNK,), jnp.int32),
            pltpu.VMEM((CHUNK, 16), jnp.float32),
            pltpu.VMEM((ZR, 16), jnp.float32),
            pltpu.VMEM_SHARED((NPAD, 16), jnp.float32),
        ],
    )
    def k(dst_hbm, out_hbm, didx_v, ones_v, zero_v, acc_sh):
        c = lax.axis_index("c")
        s = lax.axis_index("s")
        w = c * NS + s

        @pl.loop(0, CHUNK)
        def _(i):
            ones_v[i, :] = jnp.ones((16,), jnp.float32)

        @pl.loop(0, ZR)
        def _(i):
            zero_v[i, :] = jnp.zeros((16,), jnp.float32)

        base = s * RPS
        for t in range(RPS // ZR):
            pltpu.sync_copy(zero_v, acc_sh.at[pl.ds(base + t * ZR, ZR)])

        @pl.when(s == 0)
        def _():
            pltpu.sync_copy(zero_v.at[pl.ds(0, ZTAIL)],
                            acc_sh.at[pl.ds(TAIL_OFF, ZTAIL)])

        plsc.subcore_barrier()

        @pl.loop(0, CPW)
        def _(j):
            pltpu.sync_copy(dst_hbm.at[w * CPW + j], didx_v)
            pltpu.sync_copy(ones_v, acc_sh.at[didx_v], add=True)

        plsc.subcore_barrier()
        pltpu.sync_copy(acc_sh.at[pl.ds(base, RPS)],
                        out_hbm.at[c, pl.ds(base, RPS)])

        @pl.when(s == 0)
        def _():
            pltpu.sync_copy(acc_sh.at[pl.ds(TAIL_OFF, TAIL)],
                            out_hbm.at[c, pl.ds(TAIL_OFF, TAIL)])

    return k(dst2d)


def _edge_sc(hp, src2d, dst2d, zrows):
    @functools.partial(
        pl.kernel,
        out_type=jax.ShapeDtypeStruct((NC, N, D), jnp.float32),
        mesh=_sc_mesh(),
        scratch_types=[
            pltpu.VMEM((CHUNK,), jnp.int32),
            pltpu.VMEM((CHUNK,), jnp.int32),
            pltpu.VMEM((CHUNK,), jnp.int32),
            pltpu.VMEM((CHUNK,), jnp.int32),
            pltpu.VMEM((CHUNK, D), jnp.float32),
            pltpu.VMEM((CHUNK, D), jnp.float32),
            pltpu.VMEM_SHARED((NPAD, D), jnp.float32),
            pltpu.SemaphoreType.DMA,
            pltpu.SemaphoreType.DMA,
        ],
    )
    def k(hp_hbm, src_hbm, dst_hbm, z_hbm, out_hbm,
          sidx_a, sidx_b, didx_a, didx_b,
          rows_a, rows_b, acc_sh, sem_a, sem_b):
        c = lax.axis_index("c")
        s = lax.axis_index("s")
        w = c * NS + s
        base = s * RPS

        pltpu.sync_copy(z_hbm.at[pl.ds(0, RPS)], acc_sh.at[pl.ds(base, RPS)])

        @pl.when(s == 0)
        def _():
            pltpu.sync_copy(z_hbm.at[pl.ds(0, ZTAIL)],
                            acc_sh.at[pl.ds(TAIL_OFF, ZTAIL)])

        plsc.subcore_barrier()

        @pl.loop(0, CPW // 2)
        def _(p):
            ch0 = w * CPW + 2 * p
            pltpu.sync_copy(src_hbm.at[ch0], sidx_a)
            pltpu.sync_copy(dst_hbm.at[ch0], didx_a)
            pltpu.sync_copy(src_hbm.at[ch0 + 1], sidx_b)
            pltpu.sync_copy(dst_hbm.at[ch0 + 1], didx_b)
            ca = pltpu.async_copy(hp_hbm.at[sidx_a], rows_a, sem_a)
            cb = pltpu.async_copy(hp_hbm.at[sidx_b], rows_b, sem_b)
            ca.wait()
            pltpu.sync_copy(rows_a, acc_sh.at[didx_a], add=True)
            cb.wait()
            pltpu.sync_copy(rows_b, acc_sh.at[didx_b], add=True)

        plsc.subcore_barrier()
        pltpu.sync_copy(acc_sh.at[pl.ds(base, RPS)],
                        out_hbm.at[c, pl.ds(base, RPS)])

        @pl.when(s == 0)
        def _():
            pltpu.sync_copy(acc_sh.at[pl.ds(TAIL_OFF, TAIL)],
                            out_hbm.at[c, pl.ds(TAIL_OFF, TAIL)])

    return k(hp, src2d, dst2d, zrows)


def _prep_tc(xp, wep, be2, w0, dega, degb):
    def body(xp_ref, we_ref, be_ref, w0_ref, dga_ref, dgb_ref,
             atoms_ref, hp_ref, dis_ref):
        a = jnp.log(xp_ref[...] + 1.0)
        a = _dotT(a, we_ref[...]) + be_ref[...]
        deg = dga_ref[:, 0:1] + dgb_ref[:, 0:1] + 2.0
        dis = jnp.broadcast_to(lax.rsqrt(deg), (BLK, D))
        atoms_ref[...] = a
        hp_ref[...] = _dotT(a, w0_ref[...]) * dis
        dis_ref[...] = dis

    fdd = jax.ShapeDtypeStruct((N, D), jnp.float32)
    return pl.pallas_call(
        body,
        grid=(N // BLK,),
        in_specs=[
            pl.BlockSpec((BLK, 16), lambda i: (i, 0)),
            pl.BlockSpec((D, 16), lambda i: (0, 0)),
            pl.BlockSpec((1, D), lambda i: (0, 0)),
            pl.BlockSpec((D, D), lambda i: (0, 0)),
            pl.BlockSpec((BLK, 16), lambda i: (i, 0)),
            pl.BlockSpec((BLK, 16), lambda i: (i, 0)),
        ],
        out_specs=[
            pl.BlockSpec((BLK, D), lambda i: (i, 0)),
            pl.BlockSpec((BLK, D), lambda i: (i, 0)),
            pl.BlockSpec((BLK, D), lambda i: (i, 0)),
        ],
        out_shape=[fdd, fdd, fdd],
    )(xp, wep, be2, w0, dega, degb)


def _layer_tc(atoms, hp, acca, accb, dis, b2, wnext):
    def body(at_ref, hp_ref, aa_ref, ab_ref, dis_ref, b_ref, wn_ref,
             ao_ref, ho_ref):
        dis = dis_ref[...]
        out = dis * (aa_ref[...] + ab_ref[...] + 2.0 * hp_ref[...]) + b_ref[...]
        a = at_ref[...] + jnp.maximum(out, 0.0)
        ao_ref[...] = a
        ho_ref[...] = _dotT(a, wn_ref[...]) * dis

    fdd = jax.ShapeDtypeStruct((N, D), jnp.float32)
    return pl.pallas_call(
        body,
        grid=(N // BLK,),
        in_specs=[
            pl.BlockSpec((BLK, D), lambda i: (i, 0)),
            pl.BlockSpec((BLK, D), lambda i: (i, 0)),
            pl.BlockSpec((BLK, D), lambda i: (i, 0)),
            pl.BlockSpec((BLK, D), lambda i: (i, 0)),
            pl.BlockSpec((BLK, D), lambda i: (i, 0)),
            pl.BlockSpec((1, D), lambda i: (0, 0)),
            pl.BlockSpec((D, D), lambda i: (0, 0)),
        ],
        out_specs=[
            pl.BlockSpec((BLK, D), lambda i: (i, 0)),
            pl.BlockSpec((BLK, D), lambda i: (i, 0)),
        ],
        out_shape=[fdd, fdd],
    )(atoms, hp, acca, accb, dis, b2, wnext)


def _final_tc(atoms, hp, acca, accb, dis, b2, batch3d):
    def body(at_ref, hp_ref, aa_ref, ab_ref, dis_ref, b_ref, bt_ref,
             pool_ref):
        dis = dis_ref[...]
        out = dis * (aa_ref[...] + ab_ref[...] + 2.0 * hp_ref[...]) + b_ref[...]
        a = at_ref[...] + jnp.maximum(out, 0.0)
        bvec = bt_ref[0, 0, :]
        oh = (lax.broadcasted_iota(jnp.int32, (G, BLK), 0)
              == bvec[None, :]).astype(jnp.float32)
        p = jnp.dot(oh, a, precision=_PREC)

        @pl.when(pl.program_id(0) == 0)
        def _():
            pool_ref[...] = p

        @pl.when(pl.program_id(0) > 0)
        def _():
            pool_ref[...] += p

    return pl.pallas_call(
        body,
        grid=(N // BLK,),
        in_specs=[
            pl.BlockSpec((BLK, D), lambda i: (i, 0)),
            pl.BlockSpec((BLK, D), lambda i: (i, 0)),
            pl.BlockSpec((BLK, D), lambda i: (i, 0)),
            pl.BlockSpec((BLK, D), lambda i: (i, 0)),
            pl.BlockSpec((BLK, D), lambda i: (i, 0)),
            pl.BlockSpec((1, D), lambda i: (0, 0)),
            pl.BlockSpec((1, 1, BLK), lambda i: (i, 0, 0)),
        ],
        out_specs=pl.BlockSpec((G, D), lambda i: (0, 0)),
        out_shape=jax.ShapeDtypeStruct((G, D), jnp.float32),
    )(atoms, hp, acca, accb, dis, b2, batch3d)


def kernel(x, edge_index, batch, W_exp, b_exp, Wc, bc):
    pad = E2 - E
    src2d = jnp.concatenate(
        [edge_index[0].astype(jnp.int32), jnp.zeros((pad,), jnp.int32)]
    ).reshape(NCH, CHUNK)
    dst2d = jnp.concatenate(
        [edge_index[1].astype(jnp.int32), jnp.full((pad,), N, jnp.int32)]
    ).reshape(NCH, CHUNK)
    xp = jnp.pad(x.astype(jnp.float32), ((0, 0), (0, 5)))
    wep = jnp.pad(W_exp, ((0, 0), (0, 5)))
    batch3d = batch.astype(jnp.int32).reshape(N // BLK, 1, BLK)

    zrows = jnp.zeros((RPS, D), jnp.float32)

    deg = _deg_sc(dst2d)
    atoms, hp, dis = _prep_tc(xp, wep, b_exp.reshape(1, D), Wc[0],
                              deg[0], deg[1])
    for i in range(L):
        acc = _edge_sc(hp, src2d, dst2d, zrows)
        if i < L - 1:
            atoms, hp = _layer_tc(atoms, hp, acc[0], acc[1], dis,
                                  bc[i].reshape(1, D), Wc[i + 1])
        else:
            pool = _final_tc(atoms, hp, acc[0], acc[1], dis,
                             bc[i].reshape(1, D), batch3d)
    return pool


# speedup vs baseline: 1.2260x; 1.2260x over previous
import functools

import jax
import jax.numpy as jnp
from jax import lax
from jax.experimental import pallas as pl
from jax.experimental.pallas import tpu as pltpu
from jax.experimental.pallas import tpu_sc as plsc

N = 10000
E = 320000
D = 128
G = 256
L = 5

NC = 2
NS = 16
CHUNK = 128
NW = NC * NS
CPW = 80
HCPW = CPW // 2
NPH = 2
E2 = NW * CPW * CHUNK
NCH = E2 // CHUNK
NPAD = N + 16
RPS = 624
TAIL = N - NS * RPS
ZTAIL = NPAD - NS * RPS
TAIL_OFF = NS * RPS
ZR = 208

BLK = 2000
_PREC = lax.Precision.HIGHEST


def _dotT(a, w):
    return lax.dot_general(a, w, (((1,), (1,)), ((), ())), precision=_PREC)


def _extract_row(blk, j, out1d):
    for t in range(CHUNK // 16):
        out1d[pl.ds(16 * t, 16)] = blk[j, pl.ds(16 * t, 16)]


def _sc_mesh():
    return plsc.VectorSubcoreMesh(core_axis_name="c", subcore_axis_name="s")


def _deg_sc(dst2d):
    @functools.partial(
        pl.kernel,
        out_type=jax.ShapeDtypeStruct((NC, N, 16), jnp.float32),
        mesh=_sc_mesh(),
        scratch_types=[
            pltpu.VMEM((CHUNK,), jnp.int32),
            pltpu.VMEM((CHUNK, 16), jnp.float32),
            pltpu.VMEM((ZR, 16), jnp.float32),
            pltpu.VMEM_SHARED((NPAD, 16), jnp.float32),
        ],
    )
    def k(dst_hbm, out_hbm, didx_v, ones_v, zero_v, acc_sh):
        c = lax.axis_index("c")
        s = lax.axis_index("s")
        w = c * NS + s

        @pl.loop(0, CHUNK)
        def _(i):
            ones_v[i, :] = jnp.ones((16,), jnp.float32)

        @pl.loop(0, ZR)
        def _(i):
            zero_v[i, :] = jnp.zeros((16,), jnp.float32)

        base = s * RPS
        for t in range(RPS // ZR):
            pltpu.sync_copy(zero_v, acc_sh.at[pl.ds(base + t * ZR, ZR)])

        @pl.when(s == 0)
        def _():
            pltpu.sync_copy(zero_v.at[pl.ds(0, ZTAIL)],
                            acc_sh.at[pl.ds(TAIL_OFF, ZTAIL)])

        plsc.subcore_barrier()

        @pl.loop(0, CPW)
        def _(j):
            pltpu.sync_copy(dst_hbm.at[w * CPW + j], didx_v)
            pltpu.sync_copy(ones_v, acc_sh.at[didx_v], add=True)

        plsc.subcore_barrier()
        pltpu.sync_copy(acc_sh.at[pl.ds(base, RPS)],
                        out_hbm.at[c, pl.ds(base, RPS)])

        @pl.when(s == 0)
        def _():
            pltpu.sync_copy(acc_sh.at[pl.ds(TAIL_OFF, TAIL)],
                            out_hbm.at[c, pl.ds(TAIL_OFF, TAIL)])

    return k(dst2d)


def _edge_sc(hp, src2d, dst2d, zrows):
    @functools.partial(
        pl.kernel,
        out_type=jax.ShapeDtypeStruct((NC, N, D), jnp.float32),
        mesh=_sc_mesh(),
        scratch_types=[
            pltpu.VMEM((CHUNK,), jnp.int32),
            pltpu.VMEM((CHUNK,), jnp.int32),
            pltpu.VMEM((CHUNK,), jnp.int32),
            pltpu.VMEM((CHUNK,), jnp.int32),
            pltpu.VMEM((CHUNK, D), jnp.float32),
            pltpu.VMEM((CHUNK, D), jnp.float32),
            pltpu.VMEM_SHARED((NPAD, D), jnp.float32),
            pltpu.SemaphoreType.DMA,
            pltpu.SemaphoreType.DMA,
            pltpu.SemaphoreType.DMA,
            pltpu.SemaphoreType.DMA,
        ],
    )
    def k(hp_hbm, src_hbm, dst_hbm, z_hbm, out_hbm,
          sidx_a, sidx_b, didx_a, didx_b,
          rows_a, rows_b, acc_sh, sem_a, sem_b, sem_ia, sem_ib):
        c = lax.axis_index("c")
        s = lax.axis_index("s")
        w = c * NS + s
        base = s * RPS

        pltpu.sync_copy(z_hbm.at[pl.ds(0, RPS)], acc_sh.at[pl.ds(base, RPS)])

        @pl.when(s == 0)
        def _():
            pltpu.sync_copy(z_hbm.at[pl.ds(0, ZTAIL)],
                            acc_sh.at[pl.ds(TAIL_OFF, ZTAIL)])

        plsc.subcore_barrier()

        @pl.loop(0, CPW // 2)
        def _(p):
            ch0 = w * CPW + 2 * p
            i0 = pltpu.async_copy(src_hbm.at[ch0], sidx_a, sem_ia)
            i1 = pltpu.async_copy(dst_hbm.at[ch0], didx_a, sem_ia)
            i2 = pltpu.async_copy(src_hbm.at[ch0 + 1], sidx_b, sem_ib)
            i3 = pltpu.async_copy(dst_hbm.at[ch0 + 1], didx_b, sem_ib)
            i0.wait()
            i1.wait()
            i2.wait()
            i3.wait()
            ca = pltpu.async_copy(hp_hbm.at[sidx_a], rows_a, sem_a)
            cb = pltpu.async_copy(hp_hbm.at[sidx_b], rows_b, sem_b)
            ca.wait()
            pltpu.sync_copy(rows_a, acc_sh.at[didx_a], add=True)
            cb.wait()
            pltpu.sync_copy(rows_b, acc_sh.at[didx_b], add=True)

        plsc.subcore_barrier()
        pltpu.sync_copy(acc_sh.at[pl.ds(base, RPS)],
                        out_hbm.at[c, pl.ds(base, RPS)])

        @pl.when(s == 0)
        def _():
            pltpu.sync_copy(acc_sh.at[pl.ds(TAIL_OFF, TAIL)],
                            out_hbm.at[c, pl.ds(TAIL_OFF, TAIL)])

    return k(hp, src2d, dst2d, zrows)


def _prep_tc(xp, wep, be2, w0, dega, degb):
    def body(xp_ref, we_ref, be_ref, w0_ref, dga_ref, dgb_ref,
             atoms_ref, hp_ref, dis_ref):
        a = jnp.log(xp_ref[...] + 1.0)
        a = _dotT(a, we_ref[...]) + be_ref[...]
        deg = dga_ref[:, 0:1] + dgb_ref[:, 0:1] + 2.0
        dis = jnp.broadcast_to(lax.rsqrt(deg), (BLK, D))
        atoms_ref[...] = a
        hp_ref[...] = _dotT(a, w0_ref[...]) * dis
        dis_ref[...] = dis

    fdd = jax.ShapeDtypeStruct((N, D), jnp.float32)
    return pl.pallas_call(
        body,
        grid=(N // BLK,),
        in_specs=[
            pl.BlockSpec((BLK, 16), lambda i: (i, 0)),
            pl.BlockSpec((D, 16), lambda i: (0, 0)),
            pl.BlockSpec((1, D), lambda i: (0, 0)),
            pl.BlockSpec((D, D), lambda i: (0, 0)),
            pl.BlockSpec((BLK, 16), lambda i: (i, 0)),
            pl.BlockSpec((BLK, 16), lambda i: (i, 0)),
        ],
        out_specs=[
            pl.BlockSpec((BLK, D), lambda i: (i, 0)),
            pl.BlockSpec((BLK, D), lambda i: (i, 0)),
            pl.BlockSpec((BLK, D), lambda i: (i, 0)),
        ],
        out_shape=[fdd, fdd, fdd],
    )(xp, wep, be2, w0, dega, degb)


def _layer_tc(atoms, hp, acca, accb, dis, b2, wnext):
    def body(at_ref, hp_ref, aa_ref, ab_ref, dis_ref, b_ref, wn_ref,
             ao_ref, ho_ref):
        dis = dis_ref[...]
        out = dis * (aa_ref[...] + ab_ref[...] + 2.0 * hp_ref[...]) + b_ref[...]
        a = at_ref[...] + jnp.maximum(out, 0.0)
        ao_ref[...] = a
        ho_ref[...] = _dotT(a, wn_ref[...]) * dis

    fdd = jax.ShapeDtypeStruct((N, D), jnp.float32)
    return pl.pallas_call(
        body,
        grid=(N // BLK,),
        in_specs=[
            pl.BlockSpec((BLK, D), lambda i: (i, 0)),
            pl.BlockSpec((BLK, D), lambda i: (i, 0)),
            pl.BlockSpec((BLK, D), lambda i: (i, 0)),
            pl.BlockSpec((BLK, D), lambda i: (i, 0)),
            pl.BlockSpec((BLK, D), lambda i: (i, 0)),
            pl.BlockSpec((1, D), lambda i: (0, 0)),
            pl.BlockSpec((D, D), lambda i: (0, 0)),
        ],
        out_specs=[
            pl.BlockSpec((BLK, D), lambda i: (i, 0)),
            pl.BlockSpec((BLK, D), lambda i: (i, 0)),
        ],
        out_shape=[fdd, fdd],
    )(atoms, hp, acca, accb, dis, b2, wnext)


def _final_tc(atoms, hp, acca, accb, dis, b2, batch3d):
    def body(at_ref, hp_ref, aa_ref, ab_ref, dis_ref, b_ref, bt_ref,
             pool_ref):
        dis = dis_ref[...]
        out = dis * (aa_ref[...] + ab_ref[...] + 2.0 * hp_ref[...]) + b_ref[...]
        a = at_ref[...] + jnp.maximum(out, 0.0)
        bvec = bt_ref[0, 0, :]
        oh = (lax.broadcasted_iota(jnp.int32, (G, BLK), 0)
              == bvec[None, :]).astype(jnp.float32)
        p = jnp.dot(oh, a, precision=_PREC)

        @pl.when(pl.program_id(0) == 0)
        def _():
            pool_ref[...] = p

        @pl.when(pl.program_id(0) > 0)
        def _():
            pool_ref[...] += p

    return pl.pallas_call(
        body,
        grid=(N // BLK,),
        in_specs=[
            pl.BlockSpec((BLK, D), lambda i: (i, 0)),
            pl.BlockSpec((BLK, D), lambda i: (i, 0)),
            pl.BlockSpec((BLK, D), lambda i: (i, 0)),
            pl.BlockSpec((BLK, D), lambda i: (i, 0)),
            pl.BlockSpec((BLK, D), lambda i: (i, 0)),
            pl.BlockSpec((1, D), lambda i: (0, 0)),
            pl.BlockSpec((1, 1, BLK), lambda i: (i, 0, 0)),
        ],
        out_specs=pl.BlockSpec((G, D), lambda i: (0, 0)),
        out_shape=jax.ShapeDtypeStruct((G, D), jnp.float32),
    )(atoms, hp, acca, accb, dis, b2, batch3d)


def kernel(x, edge_index, batch, W_exp, b_exp, Wc, bc):
    epw = E // NW
    padw = CPW * CHUNK - epw
    src_w = edge_index[0].astype(jnp.int32).reshape(NW, epw)
    dst_w = edge_index[1].astype(jnp.int32).reshape(NW, epw)
    pad_src = jnp.zeros((NW, padw), jnp.int32)
    pad_dst = jnp.broadcast_to(
        N + (jnp.arange(padw, dtype=jnp.int32) % (NPAD - N)), (NW, padw))
    src2d = jnp.concatenate([src_w, pad_src], axis=1).reshape(NCH, CHUNK)
    dst2d = jnp.concatenate([dst_w, pad_dst], axis=1).reshape(NCH, CHUNK)
    xp = jnp.pad(x.astype(jnp.float32), ((0, 0), (0, 5)))
    wep = jnp.pad(W_exp, ((0, 0), (0, 5)))
    batch3d = batch.astype(jnp.int32).reshape(N // BLK, 1, BLK)

    zrows = jnp.zeros((RPS, D), jnp.float32)

    deg = _deg_sc(dst2d)
    atoms, hp, dis = _prep_tc(xp, wep, b_exp.reshape(1, D), Wc[0],
                              deg[0], deg[1])
    for i in range(L):
        acc = _edge_sc(hp, src2d, dst2d, zrows)
        if i < L - 1:
            atoms, hp = _layer_tc(atoms, hp, acc[0], acc[1], dis,
                                  bc[i].reshape(1, D), Wc[i + 1])
        else:
            pool = _final_tc(atoms, hp, acc[0], acc[1], dis,
                             bc[i].reshape(1, D), batch3d)
    return pool


# speedup vs baseline: 1.2388x; 1.0104x over previous
import functools

import jax
import jax.numpy as jnp
from jax import lax
from jax.experimental import pallas as pl
from jax.experimental.pallas import tpu as pltpu
from jax.experimental.pallas import tpu_sc as plsc

N = 10000
E = 320000
D = 128
G = 256
L = 5

NC = 2
NS = 16
CHUNK = 128
NW = NC * NS
CPW = 80
HCPW = CPW // 2
NPH = 2
E2 = NW * CPW * CHUNK
NCH = E2 // CHUNK
NPAD = N + 16
RPS = 624
TAIL = N - NS * RPS
ZTAIL = NPAD - NS * RPS
TAIL_OFF = NS * RPS
ZR = 208
ZRE = 104

BLK = 2000
_PREC = lax.Precision.HIGHEST


def _dotT(a, w):
    return lax.dot_general(a, w, (((1,), (1,)), ((), ())), precision=_PREC)


def _extract_row(blk, j, out1d):
    for t in range(CHUNK // 16):
        out1d[pl.ds(16 * t, 16)] = blk[j, pl.ds(16 * t, 16)]


def _sc_mesh():
    return plsc.VectorSubcoreMesh(core_axis_name="c", subcore_axis_name="s")


def _deg_sc(dst2d):
    @functools.partial(
        pl.kernel,
        out_type=jax.ShapeDtypeStruct((NC, N, 16), jnp.float32),
        mesh=_sc_mesh(),
        scratch_types=[
            pltpu.VMEM((CHUNK,), jnp.int32),
            pltpu.VMEM((CHUNK, 16), jnp.float32),
            pltpu.VMEM((ZR, 16), jnp.float32),
            pltpu.VMEM_SHARED((NPAD, 16), jnp.float32),
        ],
    )
    def k(dst_hbm, out_hbm, didx_v, ones_v, zero_v, acc_sh):
        c = lax.axis_index("c")
        s = lax.axis_index("s")
        w = c * NS + s

        @pl.loop(0, CHUNK)
        def _(i):
            ones_v[i, :] = jnp.ones((16,), jnp.float32)

        @pl.loop(0, ZR)
        def _(i):
            zero_v[i, :] = jnp.zeros((16,), jnp.float32)

        base = s * RPS
        for t in range(RPS // ZR):
            pltpu.sync_copy(zero_v, acc_sh.at[pl.ds(base + t * ZR, ZR)])

        @pl.when(s == 0)
        def _():
            pltpu.sync_copy(zero_v.at[pl.ds(0, ZTAIL)],
                            acc_sh.at[pl.ds(TAIL_OFF, ZTAIL)])

        plsc.subcore_barrier()

        @pl.loop(0, CPW)
        def _(j):
            pltpu.sync_copy(dst_hbm.at[w * CPW + j], didx_v)
            pltpu.sync_copy(ones_v, acc_sh.at[didx_v], add=True)

        plsc.subcore_barrier()
        pltpu.sync_copy(acc_sh.at[pl.ds(base, RPS)],
                        out_hbm.at[c, pl.ds(base, RPS)])

        @pl.when(s == 0)
        def _():
            pltpu.sync_copy(acc_sh.at[pl.ds(TAIL_OFF, TAIL)],
                            out_hbm.at[c, pl.ds(TAIL_OFF, TAIL)])

    return k(dst2d)


def _edge_sc(hp, src2d, dst2d):
    @functools.partial(
        pl.kernel,
        out_type=jax.ShapeDtypeStruct((NC, N, D), jnp.float32),
        mesh=_sc_mesh(),
        scratch_types=[
            pltpu.VMEM((CHUNK,), jnp.int32),
            pltpu.VMEM((CHUNK,), jnp.int32),
            pltpu.VMEM((CHUNK,), jnp.int32),
            pltpu.VMEM((CHUNK,), jnp.int32),
            pltpu.VMEM((CHUNK, D), jnp.float32),
            pltpu.VMEM((CHUNK, D), jnp.float32),
            pltpu.VMEM((ZRE, D), jnp.float32),
            pltpu.VMEM_SHARED((NPAD, D), jnp.float32),
            pltpu.SemaphoreType.DMA,
            pltpu.SemaphoreType.DMA,
            pltpu.SemaphoreType.DMA,
            pltpu.SemaphoreType.DMA,
        ],
    )
    def k(hp_hbm, src_hbm, dst_hbm, out_hbm,
          sidx_a, sidx_b, didx_a, didx_b,
          rows_a, rows_b, zero_v, acc_sh, sem_a, sem_b, sem_ia, sem_ib):
        c = lax.axis_index("c")
        s = lax.axis_index("s")
        w = c * NS + s
        base = s * RPS

        @pl.loop(0, ZRE)
        def _(i):
            @pl.loop(0, D, step=16)
            def _(j):
                zero_v[i, pl.ds(j, 16)] = jnp.zeros((16,), jnp.float32)

        for t in range(RPS // ZRE):
            pltpu.sync_copy(zero_v, acc_sh.at[pl.ds(base + t * ZRE, ZRE)])

        @pl.when(s == 0)
        def _():
            pltpu.sync_copy(zero_v.at[pl.ds(0, ZTAIL)],
                            acc_sh.at[pl.ds(TAIL_OFF, ZTAIL)])

        plsc.subcore_barrier()

        @pl.loop(0, CPW // 2)
        def _(p):
            ch0 = w * CPW + 2 * p
            i0 = pltpu.async_copy(src_hbm.at[ch0], sidx_a, sem_ia)
            i1 = pltpu.async_copy(dst_hbm.at[ch0], didx_a, sem_ia)
            i2 = pltpu.async_copy(src_hbm.at[ch0 + 1], sidx_b, sem_ib)
            i3 = pltpu.async_copy(dst_hbm.at[ch0 + 1], didx_b, sem_ib)
            i0.wait()
            i1.wait()
            i2.wait()
            i3.wait()
            ca = pltpu.async_copy(hp_hbm.at[sidx_a], rows_a, sem_a)
            cb = pltpu.async_copy(hp_hbm.at[sidx_b], rows_b, sem_b)
            ca.wait()
            pltpu.sync_copy(rows_a, acc_sh.at[didx_a], add=True)
            cb.wait()
            pltpu.sync_copy(rows_b, acc_sh.at[didx_b], add=True)

        plsc.subcore_barrier()
        pltpu.sync_copy(acc_sh.at[pl.ds(base, RPS)],
                        out_hbm.at[c, pl.ds(base, RPS)])

        @pl.when(s == 0)
        def _():
            pltpu.sync_copy(acc_sh.at[pl.ds(TAIL_OFF, TAIL)],
                            out_hbm.at[c, pl.ds(TAIL_OFF, TAIL)])

    return k(hp, src2d, dst2d)


def _prep_tc(xp, wep, be2, w0, dega, degb):
    def body(xp_ref, we_ref, be_ref, w0_ref, dga_ref, dgb_ref,
             atoms_ref, hp_ref, dis_ref):
        a = jnp.log(xp_ref[...] + 1.0)
        a = _dotT(a, we_ref[...]) + be_ref[...]
        deg = dga_ref[:, 0:1] + dgb_ref[:, 0:1] + 2.0
        dis = jnp.broadcast_to(lax.rsqrt(deg), (BLK, D))
        atoms_ref[...] = a
        hp_ref[...] = _dotT(a, w0_ref[...]) * dis
        dis_ref[...] = dis

    fdd = jax.ShapeDtypeStruct((N, D), jnp.float32)
    return pl.pallas_call(
        body,
        grid=(N // BLK,),
        in_specs=[
            pl.BlockSpec((BLK, 16), lambda i: (i, 0)),
            pl.BlockSpec((D, 16), lambda i: (0, 0)),
            pl.BlockSpec((1, D), lambda i: (0, 0)),
            pl.BlockSpec((D, D), lambda i: (0, 0)),
            pl.BlockSpec((BLK, 16), lambda i: (i, 0)),
            pl.BlockSpec((BLK, 16), lambda i: (i, 0)),
        ],
        out_specs=[
            pl.BlockSpec((BLK, D), lambda i: (i, 0)),
            pl.BlockSpec((BLK, D), lambda i: (i, 0)),
            pl.BlockSpec((BLK, D), lambda i: (i, 0)),
        ],
        out_shape=[fdd, fdd, fdd],
    )(xp, wep, be2, w0, dega, degb)


def _layer_tc(atoms, hp, acca, accb, dis, b2, wnext):
    def body(at_ref, hp_ref, aa_ref, ab_ref, dis_ref, b_ref, wn_ref,
             ao_ref, ho_ref):
        dis = dis_ref[...]
        out = dis * (aa_ref[...] + ab_ref[...] + 2.0 * hp_ref[...]) + b_ref[...]
        a = at_ref[...] + jnp.maximum(out, 0.0)
        ao_ref[...] = a
        ho_ref[...] = _dotT(a, wn_ref[...]) * dis

    fdd = jax.ShapeDtypeStruct((N, D), jnp.float32)
    return pl.pallas_call(
        body,
        grid=(N // BLK,),
        in_specs=[
            pl.BlockSpec((BLK, D), lambda i: (i, 0)),
            pl.BlockSpec((BLK, D), lambda i: (i, 0)),
            pl.BlockSpec((BLK, D), lambda i: (i, 0)),
            pl.BlockSpec((BLK, D), lambda i: (i, 0)),
            pl.BlockSpec((BLK, D), lambda i: (i, 0)),
            pl.BlockSpec((1, D), lambda i: (0, 0)),
            pl.BlockSpec((D, D), lambda i: (0, 0)),
        ],
        out_specs=[
            pl.BlockSpec((BLK, D), lambda i: (i, 0)),
            pl.BlockSpec((BLK, D), lambda i: (i, 0)),
        ],
        out_shape=[fdd, fdd],
    )(atoms, hp, acca, accb, dis, b2, wnext)


def _final_tc(atoms, hp, acca, accb, dis, b2, batch3d):
    def body(at_ref, hp_ref, aa_ref, ab_ref, dis_ref, b_ref, bt_ref,
             pool_ref):
        dis = dis_ref[...]
        out = dis * (aa_ref[...] + ab_ref[...] + 2.0 * hp_ref[...]) + b_ref[...]
        a = at_ref[...] + jnp.maximum(out, 0.0)
        bvec = bt_ref[0, 0, :]
        oh = (lax.broadcasted_iota(jnp.int32, (G, BLK), 0)
              == bvec[None, :]).astype(jnp.float32)
        p = jnp.dot(oh, a, precision=_PREC)

        @pl.when(pl.program_id(0) == 0)
        def _():
            pool_ref[...] = p

        @pl.when(pl.program_id(0) > 0)
        def _():
            pool_ref[...] += p

    return pl.pallas_call(
        body,
        grid=(N // BLK,),
        in_specs=[
            pl.BlockSpec((BLK, D), lambda i: (i, 0)),
            pl.BlockSpec((BLK, D), lambda i: (i, 0)),
            pl.BlockSpec((BLK, D), lambda i: (i, 0)),
            pl.BlockSpec((BLK, D), lambda i: (i, 0)),
            pl.BlockSpec((BLK, D), lambda i: (i, 0)),
            pl.BlockSpec((1, D), lambda i: (0, 0)),
            pl.BlockSpec((1, 1, BLK), lambda i: (i, 0, 0)),
        ],
        out_specs=pl.BlockSpec((G, D), lambda i: (0, 0)),
        out_shape=jax.ShapeDtypeStruct((G, D), jnp.float32),
    )(atoms, hp, acca, accb, dis, b2, batch3d)


def kernel(x, edge_index, batch, W_exp, b_exp, Wc, bc):
    epw = E // NW
    padw = CPW * CHUNK - epw
    src_w = edge_index[0].astype(jnp.int32).reshape(NW, epw)
    dst_w = edge_index[1].astype(jnp.int32).reshape(NW, epw)
    pad_src = jnp.zeros((NW, padw), jnp.int32)
    pad_dst = jnp.broadcast_to(
        N + (jnp.arange(padw, dtype=jnp.int32) % (NPAD - N)), (NW, padw))
    src2d = jnp.concatenate([src_w, pad_src], axis=1).reshape(NCH, CHUNK)
    dst2d = jnp.concatenate([dst_w, pad_dst], axis=1).reshape(NCH, CHUNK)
    xp = jnp.pad(x.astype(jnp.float32), ((0, 0), (0, 5)))
    wep = jnp.pad(W_exp, ((0, 0), (0, 5)))
    batch3d = batch.astype(jnp.int32).reshape(N // BLK, 1, BLK)

    deg = _deg_sc(dst2d)
    atoms, hp, dis = _prep_tc(xp, wep, b_exp.reshape(1, D), Wc[0],
                              deg[0], deg[1])
    for i in range(L):
        acc = _edge_sc(hp, src2d, dst2d)
        if i < L - 1:
            atoms, hp = _layer_tc(atoms, hp, acc[0], acc[1], dis,
                                  bc[i].reshape(1, D), Wc[i + 1])
        else:
            pool = _final_tc(atoms, hp, acc[0], acc[1], dis,
                             bc[i].reshape(1, D), batch3d)
    return pool
